# baseline (speedup 1.0000x reference)
import functools
import math

import jax
import jax.numpy as jnp
from jax import lax
from jax.experimental import pallas as pl
from jax.experimental.pallas import tpu as pltpu
from jax.experimental.pallas import tpu_sc as plsc

_EMB = 64
_SCALE = math.sqrt(_EMB)
_LANES = 16


@functools.lru_cache(maxsize=None)
def _build(B, V, D):
    NC, NS = 2, 16
    NW = NC * NS
    assert B % NW == 0
    b_per_w = B // NW
    C = 256
    assert b_per_w % (2 * C) == 0
    n_chunks = b_per_w // C
    half = n_chunks // 2

    mesh = plsc.VectorSubcoreMesh(core_axis_name="c", subcore_axis_name="s")

    @functools.partial(
        pl.kernel,
        mesh=mesh,
        out_type=jax.ShapeDtypeStruct((B, D), jnp.float32),
        scratch_types=[
            pltpu.VMEM((b_per_w,), jnp.int32),
            pltpu.VMEM((C, D), jnp.float32),
            pltpu.VMEM((C, D), jnp.float32),
            pltpu.VMEM((C, D), jnp.float32),
            pltpu.VMEM((C, D), jnp.float32),
            pltpu.SemaphoreType.DMA,
            pltpu.SemaphoreType.DMA,
            pltpu.SemaphoreType.DMA,
            pltpu.SemaphoreType.DMA,
        ],
        compiler_params=pltpu.CompilerParams(use_tc_tiling_on_sc=False),
    )
    def emb_kernel(table_hbm, tok_hbm, out_hbm, idx_v, g0, g1, s0, s1,
                   sg0, sg1, ss0, ss1):
        wid = lax.axis_index("s") * NC + lax.axis_index("c")
        base = wid * b_per_w
        gb, sb = (g0, g1), (s0, s1)
        sems_g, sems_s = (sg0, sg1), (ss0, ss1)

        pltpu.sync_copy(tok_hbm.at[pl.ds(base, b_per_w)], idx_v)

        def gather(c_off, b):
            return pltpu.make_async_copy(
                table_hbm.at[idx_v.at[pl.ds(c_off, C)]], gb[b], sems_g[b])

        def scatter(c_off, b):
            return pltpu.make_async_copy(
                sb[b], out_hbm.at[pl.ds(base + c_off, C)], sems_s[b])

        def scale(b):
            g, s = gb[b], sb[b]

            def row(r, carry):
                for j in range(D // _LANES):
                    sl = pl.ds(j * _LANES, _LANES)
                    s[r, sl] = g[r, sl] * _SCALE
                return carry

            lax.fori_loop(0, C, row, 0, unroll=4)

        gather(0 * C, 0).start()
        gather(1 * C, 1).start()

        for b in range(2):
            c_off = b * C
            gather(c_off, b).wait()
            scale(b)
            scatter(c_off, b).start()
            gather(c_off + 2 * C, b).start()

        def ring(t, carry):
            for b in range(2):
                c_off = (2 * t + b) * C
                gather(c_off, b).wait()
                scatter(c_off - 2 * C, b).wait()
                scale(b)
                scatter(c_off, b).start()
                gather(c_off + 2 * C, b).start()
            return carry

        lax.fori_loop(1, half - 1, ring, 0)

        for b in range(2):
            c_off = (n_chunks - 2 + b) * C
            gather(c_off, b).wait()
            scatter(c_off - 2 * C, b).wait()
            scale(b)
            scatter(c_off, b).start()
        for b in range(2):
            scatter((n_chunks - 2 + b) * C, b).wait()

    return emb_kernel


def kernel(tokens, table):
    B0, T = tokens.shape
    V, D = table.shape
    flat = tokens.reshape(B0 * T).astype(jnp.int32)
    out = _build(B0 * T, V, D)(table, flat)
    return out.reshape(B0, T, D)
